# Initial kernel scaffold; baseline (speedup 1.0000x reference)
import jax
import jax.numpy as jnp
from jax.experimental import pallas as pl


def kernel(x, edge_index, edge_attr, target_mp, is_fixed_mask, enc_W, enc_b, enc_ln_g, enc_ln_b, film_W1, film_b1, film_W2, film_b2, conv_Wl, conv_Wr, conv_We, conv_att, conv_b, ln_g, ln_b, dec_W1, dec_b1, dec_W2, dec_b2):
    raise NotImplementedError("write your pallas kernel here")


# speedup vs baseline: 5.1817x; 5.1817x over previous
import functools

import jax
import jax.numpy as jnp
from jax import lax
from jax.experimental import pallas as pl
from jax.experimental.pallas import tpu as pltpu
from jax.experimental.pallas import tpu_sc as plsc

_N = 10000
_E = 320000
_DIN = 6
_H = 128
_HEADS = 4
_DH = 32
_L = 4
_DE = 4

_NC = 2
_NS = 16
_NW = _NC * _NS
_EPW = _E // _NW
_CH = 80
_NCH = _EPW // _CH
_NP = 10240
_RPT = _NP // _NS

_NB = 2000
_EB = 2000
_NGRID = _N // _NB
_EGRID = _E // _EB

_sc_mesh = plsc.VectorSubcoreMesh(
    core_axis_name="c", subcore_axis_name="s", num_cores=_NC, num_subcores=_NS)


@functools.partial(
    pl.kernel,
    out_type=(jax.ShapeDtypeStruct((_E, _H), jnp.float32),
              jax.ShapeDtypeStruct((_E, _H), jnp.float32)),
    mesh=_sc_mesh,
    scratch_types=[
        pltpu.VMEM((_NCH, _CH), jnp.int32),
        pltpu.VMEM((_NCH, _CH), jnp.int32),
        pltpu.VMEM((_CH, _H), jnp.float32),
        pltpu.VMEM((_CH, _H), jnp.float32),
    ],
)
def _sc_gather(xl_hbm, xr_hbm, src_hbm, dst_hbm, g1_hbm, g2_hbm,
               src_v, dst_v, b1, b2):
    wid = lax.axis_index("s") * _NC + lax.axis_index("c")
    base = wid * _EPW
    pltpu.sync_copy(src_hbm.at[wid], src_v)
    pltpu.sync_copy(dst_hbm.at[wid], dst_v)

    def body(j, carry):
        e0 = base + j * _CH
        pltpu.sync_copy(xl_hbm.at[src_v.at[j]], b1)
        pltpu.sync_copy(b1, g1_hbm.at[pl.ds(e0, _CH)])
        pltpu.sync_copy(xr_hbm.at[dst_v.at[j]], b2)
        pltpu.sync_copy(b2, g2_hbm.at[pl.ds(e0, _CH)])
        return carry

    lax.fori_loop(0, _NCH, body, 0)


@functools.partial(
    pl.kernel,
    out_type=jax.ShapeDtypeStruct((2 * _NP, _H), jnp.float32),
    mesh=_sc_mesh,
    scratch_types=[
        pltpu.VMEM((_NCH, _CH), jnp.int32),
        pltpu.VMEM((_CH, _H), jnp.float32),
        pltpu.VMEM_SHARED((_NP, _H), jnp.float32),
    ],
)
def _sc_scatter(msg_hbm, dst_hbm, num_hbm, dst_v, mbuf, sh_num):
    cid = lax.axis_index("c")
    sid = lax.axis_index("s")
    wid = sid * _NC + cid
    r0 = sid * _RPT
    zv = jnp.zeros((16,), jnp.float32)

    def zrow(i, carry):
        for k in range(_H // 16):
            mbuf[i, pl.ds(k * 16, 16)] = zv
        return carry

    lax.fori_loop(0, _CH, zrow, 0)

    def zcp(i, carry):
        pltpu.sync_copy(mbuf, sh_num.at[pl.ds(r0 + i * _CH, _CH)])
        return carry

    lax.fori_loop(0, _RPT // _CH, zcp, 0)
    pltpu.sync_copy(dst_hbm.at[wid], dst_v)
    plsc.subcore_barrier()

    def body(j, carry):
        e0 = wid * _EPW + j * _CH
        pltpu.sync_copy(msg_hbm.at[pl.ds(e0, _CH)], mbuf)
        pltpu.sync_copy(mbuf, sh_num.at[dst_v.at[j]], add=True)
        return carry

    lax.fori_loop(0, _NCH, body, 0)
    plsc.subcore_barrier()
    out0 = cid * _NP + r0

    def ocp(i, carry):
        pltpu.sync_copy(sh_num.at[pl.ds(r0 + i * _CH, _CH)], mbuf)
        pltpu.sync_copy(mbuf, num_hbm.at[pl.ds(out0 + i * _CH, _CH)])
        return carry

    lax.fori_loop(0, _RPT // _CH, ocp, 0)


def _ln_gelu(h, g, b):
    mu = jnp.mean(h, axis=-1, keepdims=True)
    var = jnp.mean(jnp.square(h - mu), axis=-1, keepdims=True)
    return jax.nn.gelu((h - mu) / jnp.sqrt(var + 1e-5) * g + b)


def _enc_film_body(x_ref, t_ref, eW, eb, eg, ebt, fW1, fb1, fW2, fb2,
                   h_ref, g_ref, b_ref):
    h_ref[...] = _ln_gelu(x_ref[...] @ eW[...] + eb[...], eg[...], ebt[...])
    f = jax.nn.gelu(t_ref[...] * fW1[...] + fb1[...])
    film = f @ fW2[...] + fb2[...]
    g_ref[...] = film[:, :_H]
    b_ref[...] = film[:, _H:]


def _xlr_body(h_ref, wl, wr, xl_ref, xr_ref):
    hh = h_ref[...]
    xl_ref[...] = hh @ wl[...]
    xr_ref[...] = hh @ wr[...]


def _edge_body(g1_ref, g2_ref, ea_ref, we, a8, s8, msg_ref, w128_ref):
    g1 = g1_ref[...]
    t = g1 + g2_ref[...] + ea_ref[...] @ we[...]
    t = jnp.where(t >= 0, t, 0.2 * t)
    w = jnp.exp(t @ a8[...])
    wb = w @ s8[...]
    msg_ref[...] = g1 * wb
    w128_ref[...] = wb


def _upd_body(n_ref, d_ref, g_ref, bt_ref, hres_ref, cb, lg, lb, h_ref):
    num = n_ref[0] + n_ref[1]
    den = d_ref[0] + d_ref[1]
    gat = num / (den + 1e-16) + cb[...]
    hh = g_ref[...] * gat + bt_ref[...]
    h_ref[...] = _ln_gelu(hh, lg[...], lb[...]) + hres_ref[...]


def _dec_body(h_ref, x_ref, m_ref, w1, b1, w2, b2, e68, nc_ref, dl_ref):
    d1 = jax.nn.gelu(h_ref[...] @ w1[...] + b1[...])
    delta = (d1 @ w2[...] + b2[...]) * m_ref[...]
    dl_ref[...] = delta
    nc_ref[...] = x_ref[...] @ e68[...] + delta


def _full(shape):
    return pl.BlockSpec(shape, lambda i: tuple(0 for _ in shape))


def _rows(shape):
    return pl.BlockSpec(shape, lambda i: (i,) + tuple(0 for _ in shape[1:]))


def kernel(x, edge_index, edge_attr, target_mp, is_fixed_mask, enc_W, enc_b,
           enc_ln_g, enc_ln_b, film_W1, film_b1, film_W2, film_b2, conv_Wl,
           conv_Wr, conv_We, conv_att, conv_b, ln_g, ln_b, dec_W1, dec_b1,
           dec_W2, dec_b2):
    f32 = jnp.float32
    src3 = edge_index[0].reshape(_NW, _NCH, _CH)
    dst3 = edge_index[1].reshape(_NW, _NCH, _CH)
    maskf = (~is_fixed_mask).astype(f32)

    m48 = jnp.eye(8, dtype=f32)[:4]
    a8 = (conv_att[:, :, :, None] * m48[None, :, None, :]).reshape(_L, _H, 8)
    s8 = jnp.zeros((8, _H), f32).at[:4].set(
        jnp.kron(jnp.eye(4, dtype=f32), jnp.ones((1, _DH), f32)))
    e68 = jnp.eye(_DIN, 8, dtype=f32)

    h, gamma, beta = pl.pallas_call(
        _enc_film_body,
        grid=(_NGRID,),
        in_specs=[_rows((_NB, _DIN)), _rows((_NB, 1)),
                  _full((_DIN, _H)), _full((1, _H)), _full((1, _H)),
                  _full((1, _H)), _full((1, 64)), _full((1, 64)),
                  _full((64, 2 * _H)), _full((1, 2 * _H))],
        out_specs=[_rows((_NB, _H))] * 3,
        out_shape=[jax.ShapeDtypeStruct((_N, _H), f32)] * 3,
    )(x, target_mp, enc_W, enc_b.reshape(1, -1), enc_ln_g.reshape(1, -1),
      enc_ln_b.reshape(1, -1), film_W1, film_b1.reshape(1, -1),
      film_W2, film_b2.reshape(1, -1))

    xlr_call = pl.pallas_call(
        _xlr_body,
        grid=(_NGRID,),
        in_specs=[_rows((_NB, _H)), _full((_H, _H)), _full((_H, _H))],
        out_specs=[_rows((_NB, _H))] * 2,
        out_shape=[jax.ShapeDtypeStruct((_N, _H), f32)] * 2,
    )
    edge_call = pl.pallas_call(
        _edge_body,
        grid=(_EGRID,),
        in_specs=[_rows((_EB, _H)), _rows((_EB, _H)), _rows((_EB, _DE)),
                  _full((_DE, _H)), _full((_H, 8)), _full((8, _H))],
        out_specs=[_rows((_EB, _H)), _rows((_EB, _H))],
        out_shape=[jax.ShapeDtypeStruct((_E, _H), f32),
                   jax.ShapeDtypeStruct((_E, _H), f32)],
    )
    upd_call = pl.pallas_call(
        _upd_body,
        grid=(_NGRID,),
        in_specs=[pl.BlockSpec((2, _NB, _H), lambda i: (0, i, 0)),
                  pl.BlockSpec((2, _NB, _H), lambda i: (0, i, 0)),
                  _rows((_NB, _H)), _rows((_NB, _H)), _rows((_NB, _H)),
                  _full((1, _H)), _full((1, _H)), _full((1, _H))],
        out_specs=[_rows((_NB, _H))],
        out_shape=[jax.ShapeDtypeStruct((_N, _H), f32)],
    )

    for l in range(_L):
        xl, xr = xlr_call(h, conv_Wl[l], conv_Wr[l])
        g1, g2 = _sc_gather(xl, xr, src3, dst3)
        msg, w128 = edge_call(g1, g2, edge_attr, conv_We[l], a8[l], s8)
        num = _sc_scatter(msg, dst3)
        den = _sc_scatter(w128, dst3)
        (h,) = upd_call(num.reshape(2, _NP, _H), den.reshape(2, _NP, _H),
                        gamma, beta, h, conv_b[l].reshape(1, -1),
                        ln_g[l].reshape(1, -1), ln_b[l].reshape(1, -1))

    nc8, dl8 = pl.pallas_call(
        _dec_body,
        grid=(_NGRID,),
        in_specs=[_rows((_NB, _H)), _rows((_NB, _DIN)), _rows((_NB, 1)),
                  _full((_H, 64)), _full((1, 64)), _full((64, 8)),
                  _full((1, 8)), _full((_DIN, 8))],
        out_specs=[_rows((_NB, 8))] * 2,
        out_shape=[jax.ShapeDtypeStruct((_N, 8), f32)] * 2,
    )(h, x, maskf, dec_W1, dec_b1.reshape(1, -1),
      jnp.zeros((64, 8), f32).at[:, :2].set(dec_W2),
      jnp.zeros((1, 8), f32).at[0, :2].set(dec_b2), e68)

    return (nc8[:, :2], dl8[:, :2])


# speedup vs baseline: 6.0848x; 1.1743x over previous
import functools

import jax
import jax.numpy as jnp
from jax import lax
from jax.experimental import pallas as pl
from jax.experimental.pallas import tpu as pltpu
from jax.experimental.pallas import tpu_sc as plsc

_N = 10000
_E = 320000
_DIN = 6
_H = 128
_HEADS = 4
_DH = 32
_L = 4
_DE = 4

_NC = 2
_NS = 16
_NW = _NC * _NS
_EPW = _E // _NW
_CH = 80
_NCH = _EPW // _CH
_NP = 10240
_RPT = _NP // _NS

_NB = 2000
_EB = 2000
_NGRID = _N // _NB
_EGRID = _E // _EB

_sc_mesh = plsc.VectorSubcoreMesh(
    core_axis_name="c", subcore_axis_name="s", num_cores=_NC, num_subcores=_NS)


@functools.partial(
    pl.kernel,
    out_type=(jax.ShapeDtypeStruct((_E, _H), jnp.float32),
              jax.ShapeDtypeStruct((_E, _H), jnp.float32)),
    mesh=_sc_mesh,
    scratch_types=[
        pltpu.VMEM((_NCH, _CH), jnp.int32),
        pltpu.VMEM((_NCH, _CH), jnp.int32),
        pltpu.VMEM((_CH, _H), jnp.float32),
        pltpu.VMEM((_CH, _H), jnp.float32),
    ],
)
def _sc_gather(xl_hbm, xr_hbm, src_hbm, dst_hbm, g1_hbm, g2_hbm,
               src_v, dst_v, b1, b2):
    wid = lax.axis_index("s") * _NC + lax.axis_index("c")
    base = wid * _EPW
    pltpu.sync_copy(src_hbm.at[wid], src_v)
    pltpu.sync_copy(dst_hbm.at[wid], dst_v)

    def body(j, carry):
        e0 = base + j * _CH
        pltpu.sync_copy(xl_hbm.at[src_v.at[j]], b1)
        pltpu.sync_copy(b1, g1_hbm.at[pl.ds(e0, _CH)])
        pltpu.sync_copy(xr_hbm.at[dst_v.at[j]], b2)
        pltpu.sync_copy(b2, g2_hbm.at[pl.ds(e0, _CH)])
        return carry

    lax.fori_loop(0, _NCH, body, 0)


@functools.partial(
    pl.kernel,
    out_type=jax.ShapeDtypeStruct((2 * _NP, _H), jnp.float32),
    mesh=_sc_mesh,
    scratch_types=[
        pltpu.VMEM((_NCH, _CH), jnp.int32),
        pltpu.VMEM((2, _CH, _H), jnp.float32),
        pltpu.VMEM_SHARED((_NP, _H), jnp.float32),
        pltpu.SemaphoreType.DMA,
        pltpu.SemaphoreType.DMA,
    ],
)
def _sc_scatter(msg_hbm, dst_hbm, num_hbm, dst_v, mbuf, sh_num, s0, s1):
    cid = lax.axis_index("c")
    sid = lax.axis_index("s")
    wid = sid * _NC + cid
    r0 = sid * _RPT
    zv = jnp.zeros((16,), jnp.float32)

    def zrow(i, carry):
        for k in range(_H // 16):
            mbuf[0, i, pl.ds(k * 16, 16)] = zv
        return carry

    lax.fori_loop(0, _CH, zrow, 0)

    def zcp(i, carry):
        pltpu.sync_copy(mbuf.at[0], sh_num.at[pl.ds(r0 + i * _CH, _CH)])
        return carry

    lax.fori_loop(0, _RPT // _CH, zcp, 0)
    pltpu.sync_copy(dst_hbm.at[wid], dst_v)
    plsc.subcore_barrier()

    base = wid * _EPW
    sems = (s0, s1)
    pltpu.async_copy(msg_hbm.at[pl.ds(base, _CH)], mbuf.at[0], s0)
    pltpu.async_copy(msg_hbm.at[pl.ds(base + _CH, _CH)], mbuf.at[1], s1)

    def body(i, carry):
        for b in range(2):
            j = 2 * i + b

            @pl.when(j < _NCH)
            def _():
                pltpu.make_async_copy(
                    msg_hbm.at[pl.ds(base + j * _CH, _CH)],
                    mbuf.at[b], sems[b]).wait()
                pltpu.sync_copy(mbuf.at[b], sh_num.at[dst_v.at[j]], add=True)

                @pl.when(j + 2 < _NCH)
                def _():
                    pltpu.async_copy(
                        msg_hbm.at[pl.ds(base + (j + 2) * _CH, _CH)],
                        mbuf.at[b], sems[b])

        return carry

    lax.fori_loop(0, _NCH // 2 + 1, body, 0)
    plsc.subcore_barrier()
    out0 = cid * _NP + r0

    def ocp(i, carry):
        pltpu.sync_copy(sh_num.at[pl.ds(r0 + i * _CH, _CH)], mbuf.at[0])
        pltpu.sync_copy(mbuf.at[0], num_hbm.at[pl.ds(out0 + i * _CH, _CH)])
        return carry

    lax.fori_loop(0, _RPT // _CH, ocp, 0)


def _ln_gelu(h, g, b):
    mu = jnp.mean(h, axis=-1, keepdims=True)
    var = jnp.mean(jnp.square(h - mu), axis=-1, keepdims=True)
    return jax.nn.gelu((h - mu) / jnp.sqrt(var + 1e-5) * g + b)


def _enc_film_body(x_ref, t_ref, eW, eb, eg, ebt, fW1, fb1, fW2, fb2,
                   h_ref, g_ref, b_ref):
    h_ref[...] = _ln_gelu(x_ref[...] @ eW[...] + eb[...], eg[...], ebt[...])
    f = jax.nn.gelu(t_ref[...] * fW1[...] + fb1[...])
    film = f @ fW2[...] + fb2[...]
    g_ref[...] = film[:, :_H]
    b_ref[...] = film[:, _H:]


def _xlr_body(h_ref, wl, wr, xl_ref, xr_ref):
    hh = h_ref[...]
    xl_ref[...] = hh @ wl[...]
    xr_ref[...] = hh @ wr[...]


def _edge_body(g1_ref, g2_ref, ea_ref, we, a8, s8, msg_ref, w128_ref):
    g1 = g1_ref[...]
    t = g1 + g2_ref[...] + ea_ref[...] @ we[...]
    t = jnp.where(t >= 0, t, 0.2 * t)
    w = jnp.exp(t @ a8[...])
    wb = w @ s8[...]
    msg_ref[...] = g1 * wb
    w128_ref[...] = wb


def _upd_body(n_ref, d_ref, g_ref, bt_ref, hres_ref, cb, lg, lb, h_ref):
    num = n_ref[0] + n_ref[1]
    den = d_ref[0] + d_ref[1]
    gat = num / (den + 1e-16) + cb[...]
    hh = g_ref[...] * gat + bt_ref[...]
    h_ref[...] = _ln_gelu(hh, lg[...], lb[...]) + hres_ref[...]


def _dec_body(h_ref, x_ref, m_ref, w1, b1, w2, b2, e68, nc_ref, dl_ref):
    d1 = jax.nn.gelu(h_ref[...] @ w1[...] + b1[...])
    delta = (d1 @ w2[...] + b2[...]) * m_ref[...]
    dl_ref[...] = delta
    nc_ref[...] = x_ref[...] @ e68[...] + delta


def _full(shape):
    return pl.BlockSpec(shape, lambda i: tuple(0 for _ in shape))


def _rows(shape):
    return pl.BlockSpec(shape, lambda i: (i,) + tuple(0 for _ in shape[1:]))


def kernel(x, edge_index, edge_attr, target_mp, is_fixed_mask, enc_W, enc_b,
           enc_ln_g, enc_ln_b, film_W1, film_b1, film_W2, film_b2, conv_Wl,
           conv_Wr, conv_We, conv_att, conv_b, ln_g, ln_b, dec_W1, dec_b1,
           dec_W2, dec_b2):
    f32 = jnp.float32
    src3 = edge_index[0].reshape(_NW, _NCH, _CH)
    dst3 = edge_index[1].reshape(_NW, _NCH, _CH)
    maskf = (~is_fixed_mask).astype(f32)

    m48 = jnp.eye(8, dtype=f32)[:4]
    a8 = (conv_att[:, :, :, None] * m48[None, :, None, :]).reshape(_L, _H, 8)
    s8 = jnp.zeros((8, _H), f32).at[:4].set(
        jnp.kron(jnp.eye(4, dtype=f32), jnp.ones((1, _DH), f32)))
    e68 = jnp.eye(_DIN, 8, dtype=f32)

    h, gamma, beta = pl.pallas_call(
        _enc_film_body,
        grid=(_NGRID,),
        in_specs=[_rows((_NB, _DIN)), _rows((_NB, 1)),
                  _full((_DIN, _H)), _full((1, _H)), _full((1, _H)),
                  _full((1, _H)), _full((1, 64)), _full((1, 64)),
                  _full((64, 2 * _H)), _full((1, 2 * _H))],
        out_specs=[_rows((_NB, _H))] * 3,
        out_shape=[jax.ShapeDtypeStruct((_N, _H), f32)] * 3,
    )(x, target_mp, enc_W, enc_b.reshape(1, -1), enc_ln_g.reshape(1, -1),
      enc_ln_b.reshape(1, -1), film_W1, film_b1.reshape(1, -1),
      film_W2, film_b2.reshape(1, -1))

    xlr_call = pl.pallas_call(
        _xlr_body,
        grid=(_NGRID,),
        in_specs=[_rows((_NB, _H)), _full((_H, _H)), _full((_H, _H))],
        out_specs=[_rows((_NB, _H))] * 2,
        out_shape=[jax.ShapeDtypeStruct((_N, _H), f32)] * 2,
    )
    edge_call = pl.pallas_call(
        _edge_body,
        grid=(_EGRID,),
        in_specs=[_rows((_EB, _H)), _rows((_EB, _H)), _rows((_EB, _DE)),
                  _full((_DE, _H)), _full((_H, 8)), _full((8, _H))],
        out_specs=[_rows((_EB, _H)), _rows((_EB, _H))],
        out_shape=[jax.ShapeDtypeStruct((_E, _H), f32),
                   jax.ShapeDtypeStruct((_E, _H), f32)],
    )
    upd_call = pl.pallas_call(
        _upd_body,
        grid=(_NGRID,),
        in_specs=[pl.BlockSpec((2, _NB, _H), lambda i: (0, i, 0)),
                  pl.BlockSpec((2, _NB, _H), lambda i: (0, i, 0)),
                  _rows((_NB, _H)), _rows((_NB, _H)), _rows((_NB, _H)),
                  _full((1, _H)), _full((1, _H)), _full((1, _H))],
        out_specs=[_rows((_NB, _H))],
        out_shape=[jax.ShapeDtypeStruct((_N, _H), f32)],
    )

    for l in range(_L):
        xl, xr = xlr_call(h, conv_Wl[l], conv_Wr[l])
        g1, g2 = _sc_gather(xl, xr, src3, dst3)
        msg, w128 = edge_call(g1, g2, edge_attr, conv_We[l], a8[l], s8)
        num = _sc_scatter(msg, dst3)
        den = _sc_scatter(w128, dst3)
        (h,) = upd_call(num.reshape(2, _NP, _H), den.reshape(2, _NP, _H),
                        gamma, beta, h, conv_b[l].reshape(1, -1),
                        ln_g[l].reshape(1, -1), ln_b[l].reshape(1, -1))

    nc8, dl8 = pl.pallas_call(
        _dec_body,
        grid=(_NGRID,),
        in_specs=[_rows((_NB, _H)), _rows((_NB, _DIN)), _rows((_NB, 1)),
                  _full((_H, 64)), _full((1, 64)), _full((64, 8)),
                  _full((1, 8)), _full((_DIN, 8))],
        out_specs=[_rows((_NB, 8))] * 2,
        out_shape=[jax.ShapeDtypeStruct((_N, 8), f32)] * 2,
    )(h, x, maskf, dec_W1, dec_b1.reshape(1, -1),
      jnp.zeros((64, 8), f32).at[:, :2].set(dec_W2),
      jnp.zeros((1, 8), f32).at[0, :2].set(dec_b2), e68)

    return (nc8[:, :2], dl8[:, :2])


# speedup vs baseline: 7.5255x; 1.2368x over previous
import functools

import jax
import jax.numpy as jnp
from jax import lax
from jax.experimental import pallas as pl
from jax.experimental.pallas import tpu as pltpu
from jax.experimental.pallas import tpu_sc as plsc

_N = 10000
_E = 320000
_DIN = 6
_H = 128
_HEADS = 4
_DH = 32
_L = 4
_DE = 4

_NC = 2
_NS = 16
_NW = _NC * _NS
_EPW = _E // _NW
_CH = 80
_NCH = _EPW // _CH
_NP = 10240
_RPT = _NP // _NS

_NB = 2000
_EB = 2000
_NGRID = _N // _NB
_EGRID = _E // _EB

_sc_mesh = plsc.VectorSubcoreMesh(
    core_axis_name="c", subcore_axis_name="s", num_cores=_NC, num_subcores=_NS)


@functools.partial(
    pl.kernel,
    out_type=(jax.ShapeDtypeStruct((_E, _H), jnp.float32),
              jax.ShapeDtypeStruct((_E, _H), jnp.float32)),
    mesh=_sc_mesh,
    scratch_types=[
        pltpu.VMEM((_NCH, _CH), jnp.int32),
        pltpu.VMEM((_NCH, _CH), jnp.int32),
        pltpu.VMEM((4, _CH, _H), jnp.float32),
        pltpu.VMEM((4, _CH, _H), jnp.float32),
    ] + [pltpu.SemaphoreType.DMA] * 16,
)
def _sc_gather(xl_hbm, xr_hbm, src_hbm, dst_hbm, g1_hbm, g2_hbm,
               src_v, dst_v, b1, b2, *sems):
    gA, wA, gB, wB = sems[0:4], sems[4:8], sems[8:12], sems[12:16]
    wid = lax.axis_index("s") * _NC + lax.axis_index("c")
    base = wid * _EPW
    pltpu.sync_copy(src_hbm.at[wid], src_v)
    pltpu.sync_copy(dst_hbm.at[wid], dst_v)

    for j in (0, 1):
        pltpu.async_copy(xl_hbm.at[src_v.at[j]], b1.at[j], gA[j])
        pltpu.async_copy(xr_hbm.at[dst_v.at[j]], b2.at[j], gB[j])

    def body(i, carry):
        for bb in range(4):
            j = 4 * i + bb

            @pl.when(j < _NCH)
            def _():
                e0 = base + j * _CH
                pltpu.make_async_copy(xl_hbm.at[src_v.at[j]], b1.at[bb],
                                      gA[bb]).wait()
                pltpu.make_async_copy(xr_hbm.at[dst_v.at[j]], b2.at[bb],
                                      gB[bb]).wait()
                pltpu.async_copy(b1.at[bb], g1_hbm.at[pl.ds(e0, _CH)], wA[bb])
                pltpu.async_copy(b2.at[bb], g2_hbm.at[pl.ds(e0, _CH)], wB[bb])
                bn = (bb + 2) % 4

                @pl.when(j + 2 < _NCH)
                def _():
                    @pl.when(j >= 2)
                    def _():
                        e2 = base + (j - 2) * _CH
                        pltpu.make_async_copy(
                            b1.at[bn], g1_hbm.at[pl.ds(e2, _CH)], wA[bn]).wait()
                        pltpu.make_async_copy(
                            b2.at[bn], g2_hbm.at[pl.ds(e2, _CH)], wB[bn]).wait()

                    pltpu.async_copy(xl_hbm.at[src_v.at[j + 2]], b1.at[bn],
                                     gA[bn])
                    pltpu.async_copy(xr_hbm.at[dst_v.at[j + 2]], b2.at[bn],
                                     gB[bn])

        return carry

    lax.fori_loop(0, (_NCH + 3) // 4, body, 0)
    for j in range(_NCH - 4, _NCH):
        bb = j % 4
        e0 = base + j * _CH
        pltpu.make_async_copy(b1.at[bb], g1_hbm.at[pl.ds(e0, _CH)],
                              wA[bb]).wait()
        pltpu.make_async_copy(b2.at[bb], g2_hbm.at[pl.ds(e0, _CH)],
                              wB[bb]).wait()


@functools.partial(
    pl.kernel,
    out_type=(jax.ShapeDtypeStruct((2 * _NP, _H), jnp.float32),
              jax.ShapeDtypeStruct((2 * _NP, _H), jnp.float32)),
    mesh=_sc_mesh,
    scratch_types=[
        pltpu.VMEM((_NCH, _CH), jnp.int32),
        pltpu.VMEM((2, _CH, _H), jnp.float32),
        pltpu.VMEM_SHARED((_NP, _H), jnp.float32),
        pltpu.SemaphoreType.DMA,
        pltpu.SemaphoreType.DMA,
    ],
)
def _sc_scatter(msg_hbm, w_hbm, dst_hbm, num_hbm, den_hbm,
                dst_v, mbuf, sh_num, s0, s1):
    cid = lax.axis_index("c")
    sid = lax.axis_index("s")
    wid = sid * _NC + cid
    r0 = sid * _RPT
    base = wid * _EPW
    out0 = cid * _NP + r0
    sems = (s0, s1)
    zv = jnp.zeros((16,), jnp.float32)
    pltpu.sync_copy(dst_hbm.at[wid], dst_v)

    def phase(src_hbm, out_hbm):
        def zrow(i, carry):
            for k in range(_H // 16):
                mbuf[0, i, pl.ds(k * 16, 16)] = zv
            return carry

        lax.fori_loop(0, _CH, zrow, 0)

        def zcp(i, carry):
            pltpu.sync_copy(mbuf.at[0], sh_num.at[pl.ds(r0 + i * _CH, _CH)])
            return carry

        lax.fori_loop(0, _RPT // _CH, zcp, 0)
        plsc.subcore_barrier()

        pltpu.async_copy(src_hbm.at[pl.ds(base, _CH)], mbuf.at[0], s0)
        pltpu.async_copy(src_hbm.at[pl.ds(base + _CH, _CH)], mbuf.at[1], s1)

        def body(i, carry):
            for b in range(2):
                j = 2 * i + b

                @pl.when(j < _NCH)
                def _():
                    pltpu.make_async_copy(
                        src_hbm.at[pl.ds(base + j * _CH, _CH)],
                        mbuf.at[b], sems[b]).wait()
                    pltpu.sync_copy(mbuf.at[b], sh_num.at[dst_v.at[j]],
                                    add=True)

                    @pl.when(j + 2 < _NCH)
                    def _():
                        pltpu.async_copy(
                            src_hbm.at[pl.ds(base + (j + 2) * _CH, _CH)],
                            mbuf.at[b], sems[b])

            return carry

        lax.fori_loop(0, _NCH // 2 + 1, body, 0)
        plsc.subcore_barrier()

        def ocp(i, carry):
            pltpu.sync_copy(sh_num.at[pl.ds(r0 + i * _CH, _CH)], mbuf.at[0])
            pltpu.sync_copy(mbuf.at[0], out_hbm.at[pl.ds(out0 + i * _CH, _CH)])
            return carry

        lax.fori_loop(0, _RPT // _CH, ocp, 0)

    phase(msg_hbm, num_hbm)
    phase(w_hbm, den_hbm)


def _ln_gelu(h, g, b):
    mu = jnp.mean(h, axis=-1, keepdims=True)
    var = jnp.mean(jnp.square(h - mu), axis=-1, keepdims=True)
    return jax.nn.gelu((h - mu) / jnp.sqrt(var + 1e-5) * g + b)


def _enc_film_body(x_ref, t_ref, eW, eb, eg, ebt, fW1, fb1, fW2, fb2,
                   h_ref, g_ref, b_ref):
    h_ref[...] = _ln_gelu(x_ref[...] @ eW[...] + eb[...], eg[...], ebt[...])
    f = jax.nn.gelu(t_ref[...] * fW1[...] + fb1[...])
    film = f @ fW2[...] + fb2[...]
    g_ref[...] = film[:, :_H]
    b_ref[...] = film[:, _H:]


def _xlr_body(h_ref, wl, wr, xl_ref, xr_ref):
    hh = h_ref[...]
    xl_ref[...] = hh @ wl[...]
    xr_ref[...] = hh @ wr[...]


def _edge_body(g1_ref, g2_ref, ea_ref, we, a8, s8, msg_ref, w128_ref):
    g1 = g1_ref[...]
    t = g1 + g2_ref[...] + ea_ref[...] @ we[...]
    t = jnp.where(t >= 0, t, 0.2 * t)
    w = jnp.exp(t @ a8[...])
    wb = w @ s8[...]
    msg_ref[...] = g1 * wb
    w128_ref[...] = wb


def _upd_body(n_ref, d_ref, g_ref, bt_ref, hres_ref, cb, lg, lb, h_ref):
    num = n_ref[0] + n_ref[1]
    den = d_ref[0] + d_ref[1]
    gat = num / (den + 1e-16) + cb[...]
    hh = g_ref[...] * gat + bt_ref[...]
    h_ref[...] = _ln_gelu(hh, lg[...], lb[...]) + hres_ref[...]


def _dec_body(h_ref, x_ref, m_ref, w1, b1, w2, b2, e68, nc_ref, dl_ref):
    d1 = jax.nn.gelu(h_ref[...] @ w1[...] + b1[...])
    delta = (d1 @ w2[...] + b2[...]) * m_ref[...]
    dl_ref[...] = delta
    nc_ref[...] = x_ref[...] @ e68[...] + delta


def _full(shape):
    return pl.BlockSpec(shape, lambda i: tuple(0 for _ in shape))


def _rows(shape):
    return pl.BlockSpec(shape, lambda i: (i,) + tuple(0 for _ in shape[1:]))


def kernel(x, edge_index, edge_attr, target_mp, is_fixed_mask, enc_W, enc_b,
           enc_ln_g, enc_ln_b, film_W1, film_b1, film_W2, film_b2, conv_Wl,
           conv_Wr, conv_We, conv_att, conv_b, ln_g, ln_b, dec_W1, dec_b1,
           dec_W2, dec_b2):
    f32 = jnp.float32
    src3 = edge_index[0].reshape(_NW, _NCH, _CH)
    dst3 = edge_index[1].reshape(_NW, _NCH, _CH)
    maskf = (~is_fixed_mask).astype(f32)

    m48 = jnp.eye(8, dtype=f32)[:4]
    a8 = (conv_att[:, :, :, None] * m48[None, :, None, :]).reshape(_L, _H, 8)
    s8 = jnp.zeros((8, _H), f32).at[:4].set(
        jnp.kron(jnp.eye(4, dtype=f32), jnp.ones((1, _DH), f32)))
    e68 = jnp.eye(_DIN, 8, dtype=f32)

    h, gamma, beta = pl.pallas_call(
        _enc_film_body,
        grid=(_NGRID,),
        in_specs=[_rows((_NB, _DIN)), _rows((_NB, 1)),
                  _full((_DIN, _H)), _full((1, _H)), _full((1, _H)),
                  _full((1, _H)), _full((1, 64)), _full((1, 64)),
                  _full((64, 2 * _H)), _full((1, 2 * _H))],
        out_specs=[_rows((_NB, _H))] * 3,
        out_shape=[jax.ShapeDtypeStruct((_N, _H), f32)] * 3,
    )(x, target_mp, enc_W, enc_b.reshape(1, -1), enc_ln_g.reshape(1, -1),
      enc_ln_b.reshape(1, -1), film_W1, film_b1.reshape(1, -1),
      film_W2, film_b2.reshape(1, -1))

    xlr_call = pl.pallas_call(
        _xlr_body,
        grid=(_NGRID,),
        in_specs=[_rows((_NB, _H)), _full((_H, _H)), _full((_H, _H))],
        out_specs=[_rows((_NB, _H))] * 2,
        out_shape=[jax.ShapeDtypeStruct((_N, _H), f32)] * 2,
    )
    edge_call = pl.pallas_call(
        _edge_body,
        grid=(_EGRID,),
        in_specs=[_rows((_EB, _H)), _rows((_EB, _H)), _rows((_EB, _DE)),
                  _full((_DE, _H)), _full((_H, 8)), _full((8, _H))],
        out_specs=[_rows((_EB, _H)), _rows((_EB, _H))],
        out_shape=[jax.ShapeDtypeStruct((_E, _H), f32),
                   jax.ShapeDtypeStruct((_E, _H), f32)],
    )
    upd_call = pl.pallas_call(
        _upd_body,
        grid=(_NGRID,),
        in_specs=[pl.BlockSpec((2, _NB, _H), lambda i: (0, i, 0)),
                  pl.BlockSpec((2, _NB, _H), lambda i: (0, i, 0)),
                  _rows((_NB, _H)), _rows((_NB, _H)), _rows((_NB, _H)),
                  _full((1, _H)), _full((1, _H)), _full((1, _H))],
        out_specs=[_rows((_NB, _H))],
        out_shape=[jax.ShapeDtypeStruct((_N, _H), f32)],
    )

    for l in range(_L):
        xl, xr = xlr_call(h, conv_Wl[l], conv_Wr[l])
        g1, g2 = _sc_gather(xl, xr, src3, dst3)
        msg, w128 = edge_call(g1, g2, edge_attr, conv_We[l], a8[l], s8)
        num, den = _sc_scatter(msg, w128, dst3)
        (h,) = upd_call(num.reshape(2, _NP, _H), den.reshape(2, _NP, _H),
                        gamma, beta, h, conv_b[l].reshape(1, -1),
                        ln_g[l].reshape(1, -1), ln_b[l].reshape(1, -1))

    nc8, dl8 = pl.pallas_call(
        _dec_body,
        grid=(_NGRID,),
        in_specs=[_rows((_NB, _H)), _rows((_NB, _DIN)), _rows((_NB, 1)),
                  _full((_H, 64)), _full((1, 64)), _full((64, 8)),
                  _full((1, 8)), _full((_DIN, 8))],
        out_specs=[_rows((_NB, 8))] * 2,
        out_shape=[jax.ShapeDtypeStruct((_N, 8), f32)] * 2,
    )(h, x, maskf, dec_W1, dec_b1.reshape(1, -1),
      jnp.zeros((64, 8), f32).at[:, :2].set(dec_W2),
      jnp.zeros((1, 8), f32).at[0, :2].set(dec_b2), e68)

    return (nc8[:, :2], dl8[:, :2])


# speedup vs baseline: 7.7144x; 1.0251x over previous
import functools

import jax
import jax.numpy as jnp
from jax import lax
from jax.experimental import pallas as pl
from jax.experimental.pallas import tpu as pltpu
from jax.experimental.pallas import tpu_sc as plsc

_N = 10000
_E = 320000
_DIN = 6
_H = 128
_HEADS = 4
_DH = 32
_L = 4
_DE = 4

_NC = 2
_NS = 16
_NW = _NC * _NS
_EPW = _E // _NW
_CH = 80
_NCH = _EPW // _CH
_NP = 10240
_RPT = _NP // _NS
_EPW2 = _E // _NS
_NCH2 = _EPW2 // _CH

_NB = 2000
_EB = 2000
_NGRID = _N // _NB
_EGRID = _E // _EB

_sc_mesh = plsc.VectorSubcoreMesh(
    core_axis_name="c", subcore_axis_name="s", num_cores=_NC, num_subcores=_NS)


@functools.partial(
    pl.kernel,
    out_type=(jax.ShapeDtypeStruct((_E, _H), jnp.float32),
              jax.ShapeDtypeStruct((_E, _H), jnp.float32)),
    mesh=_sc_mesh,
    scratch_types=[
        pltpu.VMEM((_NCH, _CH), jnp.int32),
        pltpu.VMEM((_NCH, _CH), jnp.int32),
        pltpu.VMEM((4, _CH, _H), jnp.float32),
        pltpu.VMEM((4, _CH, _H), jnp.float32),
    ] + [pltpu.SemaphoreType.DMA] * 16,
)
def _sc_gather(xl_hbm, xr_hbm, src_hbm, dst_hbm, g1_hbm, g2_hbm,
               src_v, dst_v, b1, b2, *sems):
    gA, wA, gB, wB = sems[0:4], sems[4:8], sems[8:12], sems[12:16]
    wid = lax.axis_index("s") * _NC + lax.axis_index("c")
    base = wid * _EPW
    pltpu.sync_copy(src_hbm.at[wid], src_v)
    pltpu.sync_copy(dst_hbm.at[wid], dst_v)

    for j in (0, 1):
        pltpu.async_copy(xl_hbm.at[src_v.at[j]], b1.at[j], gA[j])
        pltpu.async_copy(xr_hbm.at[dst_v.at[j]], b2.at[j], gB[j])

    def body(i, carry):
        for bb in range(4):
            j = 4 * i + bb

            @pl.when(j < _NCH)
            def _():
                e0 = base + j * _CH
                pltpu.make_async_copy(xl_hbm.at[src_v.at[j]], b1.at[bb],
                                      gA[bb]).wait()
                pltpu.make_async_copy(xr_hbm.at[dst_v.at[j]], b2.at[bb],
                                      gB[bb]).wait()
                pltpu.async_copy(b1.at[bb], g1_hbm.at[pl.ds(e0, _CH)], wA[bb])
                pltpu.async_copy(b2.at[bb], g2_hbm.at[pl.ds(e0, _CH)], wB[bb])
                bn = (bb + 2) % 4

                @pl.when(j + 2 < _NCH)
                def _():
                    @pl.when(j >= 2)
                    def _():
                        e2 = base + (j - 2) * _CH
                        pltpu.make_async_copy(
                            b1.at[bn], g1_hbm.at[pl.ds(e2, _CH)], wA[bn]).wait()
                        pltpu.make_async_copy(
                            b2.at[bn], g2_hbm.at[pl.ds(e2, _CH)], wB[bn]).wait()

                    pltpu.async_copy(xl_hbm.at[src_v.at[j + 2]], b1.at[bn],
                                     gA[bn])
                    pltpu.async_copy(xr_hbm.at[dst_v.at[j + 2]], b2.at[bn],
                                     gB[bn])

        return carry

    lax.fori_loop(0, (_NCH + 3) // 4, body, 0)
    for j in range(_NCH - 4, _NCH):
        bb = j % 4
        e0 = base + j * _CH
        pltpu.make_async_copy(b1.at[bb], g1_hbm.at[pl.ds(e0, _CH)],
                              wA[bb]).wait()
        pltpu.make_async_copy(b2.at[bb], g2_hbm.at[pl.ds(e0, _CH)],
                              wB[bb]).wait()


@functools.partial(
    pl.kernel,
    out_type=(jax.ShapeDtypeStruct((_NP, _H), jnp.float32),
              jax.ShapeDtypeStruct((_NP, _H), jnp.float32)),
    mesh=_sc_mesh,
    scratch_types=[
        pltpu.VMEM((_NCH, _CH), jnp.int32),
        pltpu.VMEM((2, _CH, _H), jnp.float32),
        pltpu.VMEM_SHARED((_NP, _H), jnp.float32),
        pltpu.SemaphoreType.DMA,
        pltpu.SemaphoreType.DMA,
    ],
)
def _sc_scatter(msg_hbm, w_hbm, dst_hbm, num_hbm, den_hbm,
                dst_v, mbuf, sh_acc, s0, s1):
    cid = lax.axis_index("c")
    sid = lax.axis_index("s")
    r0 = sid * _RPT
    base = sid * _EPW2
    sems = (s0, s1)
    zv = jnp.zeros((16,), jnp.float32)
    pltpu.sync_copy(dst_hbm.at[2 * sid], dst_v)

    def zrow(i, carry):
        for k in range(_H // 16):
            mbuf[0, i, pl.ds(k * 16, 16)] = zv
        return carry

    lax.fori_loop(0, _CH, zrow, 0)

    def zcp(i, carry):
        pltpu.sync_copy(mbuf.at[0], sh_acc.at[pl.ds(r0 + i * _CH, _CH)])
        return carry

    lax.fori_loop(0, _RPT // _CH, zcp, 0)
    plsc.subcore_barrier()

    def phase(src_hbm, out_hbm):
        pltpu.async_copy(src_hbm.at[pl.ds(base, _CH)], mbuf.at[0], s0)
        pltpu.async_copy(src_hbm.at[pl.ds(base + _CH, _CH)], mbuf.at[1], s1)

        def body(i, carry):
            for b in range(2):
                j = 2 * i + b

                @pl.when(j == _NCH)
                def _():
                    pltpu.sync_copy(dst_hbm.at[2 * sid + 1], dst_v)

                jj = lax.rem(j, _NCH)

                @pl.when(j < _NCH2)
                def _():
                    pltpu.make_async_copy(
                        src_hbm.at[pl.ds(base + j * _CH, _CH)],
                        mbuf.at[b], sems[b]).wait()
                    pltpu.sync_copy(mbuf.at[b], sh_acc.at[dst_v.at[jj]],
                                    add=True)

                    @pl.when(j + 2 < _NCH2)
                    def _():
                        pltpu.async_copy(
                            src_hbm.at[pl.ds(base + (j + 2) * _CH, _CH)],
                            mbuf.at[b], sems[b])

            return carry

        lax.fori_loop(0, _NCH2 // 2 + 1, body, 0)
        plsc.subcore_barrier()

        def ocp(i, carry):
            pltpu.sync_copy(sh_acc.at[pl.ds(r0 + i * _CH, _CH)], mbuf.at[0])
            pltpu.sync_copy(mbuf.at[0], out_hbm.at[pl.ds(r0 + i * _CH, _CH)])
            return carry

        lax.fori_loop(0, _RPT // _CH, ocp, 0)

    @pl.when(cid == 0)
    def _():
        phase(msg_hbm, num_hbm)

    @pl.when(cid == 1)
    def _():
        phase(w_hbm, den_hbm)


def _ln_gelu(h, g, b):
    mu = jnp.mean(h, axis=-1, keepdims=True)
    var = jnp.mean(jnp.square(h - mu), axis=-1, keepdims=True)
    return jax.nn.gelu((h - mu) / jnp.sqrt(var + 1e-5) * g + b)


def _enc_film_body(x_ref, t_ref, eW, eb, eg, ebt, fW1, fb1, fW2, fb2, wl, wr,
                   h_ref, g_ref, b_ref, xl_ref, xr_ref):
    hh = _ln_gelu(x_ref[...] @ eW[...] + eb[...], eg[...], ebt[...])
    h_ref[...] = hh
    xl_ref[...] = hh @ wl[...]
    xr_ref[...] = hh @ wr[...]
    f = jax.nn.gelu(t_ref[...] * fW1[...] + fb1[...])
    film = f @ fW2[...] + fb2[...]
    g_ref[...] = film[:, :_H]
    b_ref[...] = film[:, _H:]


def _edge_body(g1_ref, g2_ref, ea_ref, we, a8, s8, msg_ref, w128_ref):
    g1 = g1_ref[...]
    t = g1 + g2_ref[...] + ea_ref[...] @ we[...]
    t = jnp.where(t >= 0, t, 0.2 * t)
    w = jnp.exp(t @ a8[...])
    wb = w @ s8[...]
    msg_ref[...] = g1 * wb
    w128_ref[...] = wb


def _upd_body(n_ref, d_ref, g_ref, bt_ref, hres_ref, cb, lg, lb, h_ref):
    gat = n_ref[...] / (d_ref[...] + 1e-16) + cb[...]
    hh = g_ref[...] * gat + bt_ref[...]
    h_ref[...] = _ln_gelu(hh, lg[...], lb[...]) + hres_ref[...]


def _upd_xlr_body(n_ref, d_ref, g_ref, bt_ref, hres_ref, cb, lg, lb, wl, wr,
                  h_ref, xl_ref, xr_ref):
    gat = n_ref[...] / (d_ref[...] + 1e-16) + cb[...]
    hh = g_ref[...] * gat + bt_ref[...]
    hn = _ln_gelu(hh, lg[...], lb[...]) + hres_ref[...]
    h_ref[...] = hn
    xl_ref[...] = hn @ wl[...]
    xr_ref[...] = hn @ wr[...]


def _dec_body(h_ref, x_ref, m_ref, w1, b1, w2, b2, e68, nc_ref, dl_ref):
    d1 = jax.nn.gelu(h_ref[...] @ w1[...] + b1[...])
    delta = (d1 @ w2[...] + b2[...]) * m_ref[...]
    dl_ref[...] = delta
    nc_ref[...] = x_ref[...] @ e68[...] + delta


def _full(shape):
    return pl.BlockSpec(shape, lambda i: tuple(0 for _ in shape))


def _rows(shape):
    return pl.BlockSpec(shape, lambda i: (i,) + tuple(0 for _ in shape[1:]))


def kernel(x, edge_index, edge_attr, target_mp, is_fixed_mask, enc_W, enc_b,
           enc_ln_g, enc_ln_b, film_W1, film_b1, film_W2, film_b2, conv_Wl,
           conv_Wr, conv_We, conv_att, conv_b, ln_g, ln_b, dec_W1, dec_b1,
           dec_W2, dec_b2):
    f32 = jnp.float32
    src3 = edge_index[0].reshape(_NW, _NCH, _CH)
    dst3 = edge_index[1].reshape(_NW, _NCH, _CH)
    maskf = (~is_fixed_mask).astype(f32)

    m48 = jnp.eye(8, dtype=f32)[:4]
    a8 = (conv_att[:, :, :, None] * m48[None, :, None, :]).reshape(_L, _H, 8)
    s8 = jnp.zeros((8, _H), f32).at[:4].set(
        jnp.kron(jnp.eye(4, dtype=f32), jnp.ones((1, _DH), f32)))
    e68 = jnp.eye(_DIN, 8, dtype=f32)

    h, gamma, beta, xl, xr = pl.pallas_call(
        _enc_film_body,
        grid=(_NGRID,),
        in_specs=[_rows((_NB, _DIN)), _rows((_NB, 1)),
                  _full((_DIN, _H)), _full((1, _H)), _full((1, _H)),
                  _full((1, _H)), _full((1, 64)), _full((1, 64)),
                  _full((64, 2 * _H)), _full((1, 2 * _H)),
                  _full((_H, _H)), _full((_H, _H))],
        out_specs=[_rows((_NB, _H))] * 5,
        out_shape=[jax.ShapeDtypeStruct((_N, _H), f32)] * 5,
    )(x, target_mp, enc_W, enc_b.reshape(1, -1), enc_ln_g.reshape(1, -1),
      enc_ln_b.reshape(1, -1), film_W1, film_b1.reshape(1, -1),
      film_W2, film_b2.reshape(1, -1), conv_Wl[0], conv_Wr[0])

    edge_call = pl.pallas_call(
        _edge_body,
        grid=(_EGRID,),
        in_specs=[_rows((_EB, _H)), _rows((_EB, _H)), _rows((_EB, _DE)),
                  _full((_DE, _H)), _full((_H, 8)), _full((8, _H))],
        out_specs=[_rows((_EB, _H)), _rows((_EB, _H))],
        out_shape=[jax.ShapeDtypeStruct((_E, _H), f32),
                   jax.ShapeDtypeStruct((_E, _H), f32)],
    )
    upd_call = pl.pallas_call(
        _upd_body,
        grid=(_NGRID,),
        in_specs=[_rows((_NB, _H)), _rows((_NB, _H)),
                  _rows((_NB, _H)), _rows((_NB, _H)), _rows((_NB, _H)),
                  _full((1, _H)), _full((1, _H)), _full((1, _H))],
        out_specs=[_rows((_NB, _H))],
        out_shape=[jax.ShapeDtypeStruct((_N, _H), f32)],
    )
    upd_xlr_call = pl.pallas_call(
        _upd_xlr_body,
        grid=(_NGRID,),
        in_specs=[_rows((_NB, _H)), _rows((_NB, _H)),
                  _rows((_NB, _H)), _rows((_NB, _H)), _rows((_NB, _H)),
                  _full((1, _H)), _full((1, _H)), _full((1, _H)),
                  _full((_H, _H)), _full((_H, _H))],
        out_specs=[_rows((_NB, _H))] * 3,
        out_shape=[jax.ShapeDtypeStruct((_N, _H), f32)] * 3,
    )

    for l in range(_L):
        g1, g2 = _sc_gather(xl, xr, src3, dst3)
        msg, w128 = edge_call(g1, g2, edge_attr, conv_We[l], a8[l], s8)
        num, den = _sc_scatter(msg, w128, dst3)
        args = (num, den, gamma, beta, h,
                conv_b[l].reshape(1, -1), ln_g[l].reshape(1, -1),
                ln_b[l].reshape(1, -1))
        if l < _L - 1:
            h, xl, xr = upd_xlr_call(*args, conv_Wl[l + 1], conv_Wr[l + 1])
        else:
            (h,) = upd_call(*args)

    nc8, dl8 = pl.pallas_call(
        _dec_body,
        grid=(_NGRID,),
        in_specs=[_rows((_NB, _H)), _rows((_NB, _DIN)), _rows((_NB, 1)),
                  _full((_H, 64)), _full((1, 64)), _full((64, 8)),
                  _full((1, 8)), _full((_DIN, 8))],
        out_specs=[_rows((_NB, 8))] * 2,
        out_shape=[jax.ShapeDtypeStruct((_N, 8), f32)] * 2,
    )(h, x, maskf, dec_W1, dec_b1.reshape(1, -1),
      jnp.zeros((64, 8), f32).at[:, :2].set(dec_W2),
      jnp.zeros((1, 8), f32).at[0, :2].set(dec_b2), e68)

    return (nc8[:, :2], dl8[:, :2])
